# Initial kernel scaffold; baseline (speedup 1.0000x reference)
import jax
import jax.numpy as jnp
from jax.experimental import pallas as pl


def kernel(x, edge_index, edge_attr, edge_world_index, edge_world_attr, emb_W1, emb_b1, emb_W2, emb_b2, emb_g, emb_be, ewb_W1, ewb_b1, ewb_W2, ewb_b2, ewb_g, ewb_be, nb_W1, nb_b1, nb_W2, nb_b2, nb_g, nb_be):
    raise NotImplementedError("write your pallas kernel here")


# speedup vs baseline: 2.6752x; 2.6752x over previous
import functools

import jax
import jax.numpy as jnp
from jax import lax
from jax.experimental import pallas as pl
from jax.experimental.pallas import tpu as pltpu
from jax.experimental.pallas import tpu_sc as plsc

H = 128
NCORES = 2
NSUB = 16
NW = NCORES * NSUB


def _pre_body(x_ref, w_ref, b_ref, pam_ref, pbm_ref, paw_ref, pbw_ref):
    p = jnp.dot(x_ref[...], w_ref[...], preferred_element_type=jnp.float32)
    p = p + b_ref[...]
    pam_ref[...] = p[:, 0 * H:1 * H]
    pbm_ref[...] = p[:, 1 * H:2 * H]
    paw_ref[...] = p[:, 2 * H:3 * H]
    pbw_ref[...] = p[:, 3 * H:4 * H]


def _precompute_tables(x, wcat, bcat, bn=2000):
    n = x.shape[0]
    out = [jax.ShapeDtypeStruct((n, H), jnp.float32)] * 4
    return pl.pallas_call(
        _pre_body,
        grid=(n // bn,),
        in_specs=[
            pl.BlockSpec((bn, H), lambda i: (i, 0)),
            pl.BlockSpec((H, 4 * H), lambda i: (0, 0)),
            pl.BlockSpec((1, 4 * H), lambda i: (0, 0)),
        ],
        out_specs=[pl.BlockSpec((bn, H), lambda i: (i, 0))] * 4,
        out_shape=out,
    )(x, wcat, bcat)


def _make_gather_sum(n_edges, chunk):
    per_w = n_edges // NW
    iters = per_w // chunk
    assert per_w * NW == n_edges and iters * chunk == per_w and chunk % 8 == 0

    def body(ta_hbm, tb_hbm, src_hbm, dst_hbm, out_hbm,
             idxa, idxb, bufa, bufb, sema, semb):
        cid = lax.axis_index("c")
        sid = lax.axis_index("s")
        wid = sid * NCORES + cid
        base0 = wid * per_w

        def step(i, carry):
            base = base0 + i * chunk
            pltpu.sync_copy(src_hbm.at[pl.ds(base, chunk)], idxa)
            pltpu.sync_copy(dst_hbm.at[pl.ds(base, chunk)], idxb)
            pltpu.async_copy(ta_hbm.at[idxa], bufa, sema).wait()
            pltpu.async_copy(tb_hbm.at[idxb], bufb, semb).wait()

            def add_row(r, c):
                for h in range(H // 16):
                    sl = pl.ds(h * 16, 16)
                    bufa[r, sl] = bufa[r, sl] + bufb[r, sl]
                return c

            lax.fori_loop(0, chunk, add_row, 0)
            pltpu.sync_copy(bufa, out_hbm.at[pl.ds(base, chunk)])
            return carry

        lax.fori_loop(0, iters, step, 0)

    mesh = plsc.VectorSubcoreMesh(core_axis_name="c", subcore_axis_name="s")
    return pl.kernel(
        body,
        out_type=jax.ShapeDtypeStruct((n_edges, H), jnp.float32),
        mesh=mesh,
        scratch_types=[
            pltpu.VMEM((chunk,), jnp.int32),
            pltpu.VMEM((chunk,), jnp.int32),
            pltpu.VMEM((chunk, H), jnp.float32),
            pltpu.VMEM((chunk, H), jnp.float32),
            pltpu.SemaphoreType.DMA,
            pltpu.SemaphoreType.DMA,
        ],
    )


def _make_segment_sum(n_edges, n_nodes_pad, chunk):
    per_core = n_edges // NCORES
    per_w = per_core // NSUB
    iters = per_w // chunk
    rows_per_tile = n_nodes_pad // NSUB
    assert iters * chunk == per_w and chunk % 8 == 0 and rows_per_tile % 8 == 0

    def body(rows_hbm, dst_hbm, zeros_hbm, parts_hbm, idxv, buf, acc, sem):
        cid = lax.axis_index("c")
        sid = lax.axis_index("s")
        r0 = sid * rows_per_tile
        pltpu.sync_copy(zeros_hbm.at[pl.ds(r0, rows_per_tile)],
                        acc.at[pl.ds(r0, rows_per_tile)])
        plsc.subcore_barrier()

        base0 = cid * per_core + sid * per_w

        def step(i, carry):
            base = base0 + i * chunk
            pltpu.sync_copy(dst_hbm.at[pl.ds(base, chunk)], idxv)
            pltpu.sync_copy(rows_hbm.at[pl.ds(base, chunk)], buf)
            pltpu.sync_copy(buf, acc.at[idxv], add=True)
            return carry

        lax.fori_loop(0, iters, step, 0)
        plsc.subcore_barrier()
        pltpu.sync_copy(acc.at[pl.ds(r0, rows_per_tile)],
                        parts_hbm.at[cid, pl.ds(r0, rows_per_tile)])

    mesh = plsc.VectorSubcoreMesh(core_axis_name="c", subcore_axis_name="s")
    return pl.kernel(
        body,
        out_type=jax.ShapeDtypeStruct((NCORES, n_nodes_pad, H), jnp.float32),
        mesh=mesh,
        scratch_types=[
            pltpu.VMEM((chunk,), jnp.int32),
            pltpu.VMEM((chunk, H), jnp.float32),
            pltpu.VMEM_SHARED((n_nodes_pad, H), jnp.float32),
            pltpu.SemaphoreType.DMA,
        ],
    )


def _edge_body(g_ref, a_ref, w1c_ref, w2_ref, b2_ref, gam_ref, bet_ref,
               enew_ref, eout_ref):
    a = a_ref[...]
    t = g_ref[...] + jnp.dot(a, w1c_ref[...], preferred_element_type=jnp.float32)
    h = jnp.maximum(t, 0.0)
    y = jnp.dot(h, w2_ref[...], preferred_element_type=jnp.float32) + b2_ref[...]
    m = jnp.mean(y, axis=-1, keepdims=True)
    d = y - m
    v = jnp.mean(d * d, axis=-1, keepdims=True)
    e = d * lax.rsqrt(v + 1e-5) * gam_ref[...] + bet_ref[...]
    enew_ref[...] = e
    eout_ref[...] = a + e


def _edge_mlp(g, attr, w1c, w2, b2, gam, bet, be=2000):
    n = g.shape[0]
    out = [jax.ShapeDtypeStruct((n, H), jnp.float32)] * 2
    row = pl.BlockSpec((be, H), lambda i: (i, 0))
    full = pl.BlockSpec((H, H), lambda i: (0, 0))
    vec = pl.BlockSpec((1, H), lambda i: (0, 0))
    return pl.pallas_call(
        _edge_body,
        grid=(n // be,),
        in_specs=[row, row, full, full, vec, vec, vec],
        out_specs=[row, row],
        out_shape=out,
    )(g, attr, w1c, w2, b2, gam, bet)


def _node_body(x_ref, pm_ref, pw_ref, w1_ref, b1_ref, w2_ref, b2_ref,
               gam_ref, bet_ref, out_ref):
    x = x_ref[...]
    aggm = pm_ref[0] + pm_ref[1]
    aggw = pw_ref[0] + pw_ref[1]
    cat = jnp.concatenate([x, aggm, aggw], axis=-1)
    t = jnp.dot(cat, w1_ref[...], preferred_element_type=jnp.float32) + b1_ref[...]
    h = jnp.maximum(t, 0.0)
    y = jnp.dot(h, w2_ref[...], preferred_element_type=jnp.float32) + b2_ref[...]
    m = jnp.mean(y, axis=-1, keepdims=True)
    d = y - m
    v = jnp.mean(d * d, axis=-1, keepdims=True)
    out_ref[...] = x + d * lax.rsqrt(v + 1e-5) * gam_ref[...] + bet_ref[...]


def _node_mlp(x, parts_m, parts_w, w1, b1, w2, b2, gam, bet, bn=2000):
    n = x.shape[0]
    row = pl.BlockSpec((bn, H), lambda i: (i, 0))
    prow = pl.BlockSpec((NCORES, bn, H), lambda i: (0, i, 0))
    vec = pl.BlockSpec((1, H), lambda i: (0, 0))
    return pl.pallas_call(
        _node_body,
        grid=(n // bn,),
        in_specs=[row, prow, prow,
                  pl.BlockSpec((3 * H, H), lambda i: (0, 0)), vec,
                  pl.BlockSpec((H, H), lambda i: (0, 0)), vec, vec, vec],
        out_specs=row,
        out_shape=jax.ShapeDtypeStruct((n, H), jnp.float32),
    )(x, parts_m, parts_w, w1, b1, w2, b2, gam, bet)


def kernel(x, edge_index, edge_attr, edge_world_index, edge_world_attr,
           emb_W1, emb_b1, emb_W2, emb_b2, emb_g, emb_be,
           ewb_W1, ewb_b1, ewb_W2, ewb_b2, ewb_g, ewb_be,
           nb_W1, nb_b1, nb_W2, nb_b2, nb_g, nb_be):
    n, _ = x.shape
    e = edge_attr.shape[0]
    ew = edge_world_attr.shape[0]

    wcat = jnp.concatenate(
        [emb_W1[0:H], emb_W1[H:2 * H], ewb_W1[0:H], ewb_W1[H:2 * H]], axis=1)
    zcol = jnp.zeros((H,), jnp.float32)
    bcat = jnp.concatenate([zcol, emb_b1, zcol, ewb_b1]).reshape(1, 4 * H)
    pam, pbm, paw, pbw = _precompute_tables(x, wcat, bcat)

    src_m, dst_m = edge_index[0], edge_index[1]
    src_w, dst_w = edge_world_index[0], edge_world_index[1]

    g_m = _make_gather_sum(e, 80)(pam, pbm, src_m, dst_m)
    g_w = _make_gather_sum(ew, 40)(paw, pbw, src_w, dst_w)

    v1 = lambda a: a.reshape(1, H)
    enew_m, eout_m = _edge_mlp(g_m, edge_attr, emb_W1[2 * H:], emb_W2,
                               v1(emb_b2), v1(emb_g), v1(emb_be))
    enew_w, eout_w = _edge_mlp(g_w, edge_world_attr, ewb_W1[2 * H:], ewb_W2,
                               v1(ewb_b2), v1(ewb_g), v1(ewb_be))

    n_pad = ((n + NSUB * 8 - 1) // (NSUB * 8)) * (NSUB * 8)
    zeros_nh = jnp.zeros((n_pad, H), jnp.float32)
    parts_m = _make_segment_sum(e, n_pad, 80)(enew_m, dst_m, zeros_nh)
    parts_w = _make_segment_sum(ew, n_pad, 40)(enew_w, dst_w, zeros_nh)

    x_out = _node_mlp(x, parts_m, parts_w, nb_W1, v1(nb_b1), nb_W2,
                      v1(nb_b2), v1(nb_g), v1(nb_be))
    return (x_out, eout_m, eout_w)


# speedup vs baseline: 5.3781x; 2.0104x over previous
import functools

import jax
import jax.numpy as jnp
from jax import lax
from jax.experimental import pallas as pl
from jax.experimental.pallas import tpu as pltpu
from jax.experimental.pallas import tpu_sc as plsc

H = 128
NCORES = 2
NSUB = 16
NW = NCORES * NSUB


def _pre_body(x_ref, w_ref, b_ref, pam_ref, pbm_ref, paw_ref, pbw_ref):
    p = jnp.dot(x_ref[...], w_ref[...], preferred_element_type=jnp.float32)
    p = p + b_ref[...]
    pam_ref[...] = p[:, 0 * H:1 * H]
    pbm_ref[...] = p[:, 1 * H:2 * H]
    paw_ref[...] = p[:, 2 * H:3 * H]
    pbw_ref[...] = p[:, 3 * H:4 * H]


def _precompute_tables(x, wcat, bcat, bn=2000):
    n = x.shape[0]
    out = [jax.ShapeDtypeStruct((n, H), jnp.float32)] * 4
    return pl.pallas_call(
        _pre_body,
        grid=(n // bn,),
        in_specs=[
            pl.BlockSpec((bn, H), lambda i: (i, 0)),
            pl.BlockSpec((H, 4 * H), lambda i: (0, 0)),
            pl.BlockSpec((1, 4 * H), lambda i: (0, 0)),
        ],
        out_specs=[pl.BlockSpec((bn, H), lambda i: (i, 0))] * 4,
        out_shape=out,
    )(x, wcat, bcat)


def _make_gather_sum(n_edges, chunk, nslots):
    per_w = n_edges // NW
    iters = per_w // chunk
    assert per_w * NW == n_edges and iters * chunk == per_w
    assert chunk % 8 == 0 and iters % nslots == 0
    K = nslots

    def body(ta_hbm, tb_hbm, src_hbm, dst_hbm, out_hbm, *scr):
        idxa = scr[0:K]
        idxb = scr[K:2 * K]
        bufa = scr[2 * K:3 * K]
        bufb = scr[3 * K:4 * K]
        sgi = scr[4 * K:5 * K]
        sga = scr[5 * K:6 * K]
        sgb = scr[6 * K:7 * K]
        sso = scr[7 * K:8 * K]
        cid = lax.axis_index("c")
        sid = lax.axis_index("s")
        wid = sid * NCORES + cid
        base0 = wid * per_w

        def fire_idx(slot, i):
            sl = pl.ds(base0 + i * chunk, chunk)
            pltpu.async_copy(src_hbm.at[sl], idxa[slot], sgi[slot])
            pltpu.async_copy(dst_hbm.at[sl], idxb[slot], sgi[slot])

        def wait_idx(slot):
            pltpu.make_async_copy(src_hbm.at[pl.ds(0, chunk)], idxa[slot],
                                  sgi[slot]).wait()
            pltpu.make_async_copy(dst_hbm.at[pl.ds(0, chunk)], idxb[slot],
                                  sgi[slot]).wait()

        def fire_g(slot):
            pltpu.async_copy(ta_hbm.at[idxa[slot]], bufa[slot], sga[slot])
            pltpu.async_copy(tb_hbm.at[idxb[slot]], bufb[slot], sgb[slot])

        def wait_g(slot):
            pltpu.make_async_copy(ta_hbm.at[idxa[slot]], bufa[slot],
                                  sga[slot]).wait()
            pltpu.make_async_copy(tb_hbm.at[idxb[slot]], bufb[slot],
                                  sgb[slot]).wait()

        def wait_o(slot):
            pltpu.make_async_copy(bufa[slot], out_hbm.at[pl.ds(0, chunk)],
                                  sso[slot]).wait()

        for p in range(K):
            fire_idx(p, p)
        for p in range(K - 1):
            wait_idx(p)
            fire_g(p)

        def group(g, carry):
            for b in range(K):
                i = g * K + b
                wait_g(b)

                @pl.when(i + K < iters)
                def _nexti():
                    fire_idx(b, i + K)

                def add_row(r, c):
                    for h in range(H // 16):
                        sl = pl.ds(h * 16, 16)
                        bufa[b][r, sl] = bufa[b][r, sl] + bufb[b][r, sl]
                    return c

                lax.fori_loop(0, chunk, add_row, 0)
                pltpu.async_copy(
                    bufa[b], out_hbm.at[pl.ds(base0 + i * chunk, chunk)],
                    sso[b])
                j = i + K - 1
                s2 = (b + K - 1) % K

                @pl.when(j < iters)
                def _prefetch():
                    @pl.when(i > 0)
                    def _drain():
                        wait_o(s2)
                    wait_idx(s2)
                    fire_g(s2)
            return carry

        lax.fori_loop(0, iters // K, group, 0)
        for b in range(K):
            wait_o(b)

    mesh = plsc.VectorSubcoreMesh(core_axis_name="c", subcore_axis_name="s")
    scratch = [pltpu.VMEM((chunk,), jnp.int32)] * (2 * K)
    scratch += [pltpu.VMEM((chunk, H), jnp.float32)] * (2 * K)
    scratch += [pltpu.SemaphoreType.DMA] * (4 * K)
    return pl.kernel(
        body,
        out_type=jax.ShapeDtypeStruct((n_edges, H), jnp.float32),
        mesh=mesh,
        scratch_types=scratch,
    )


def _make_segment_sum(n_edges, n_nodes_pad, chunk, nslots):
    per_core = n_edges // NCORES
    per_w = per_core // NSUB
    iters = per_w // chunk
    rows_per_tile = n_nodes_pad // NSUB
    assert iters * chunk == per_w and chunk % 8 == 0 and rows_per_tile % 8 == 0
    assert iters % nslots == 0
    K = nslots

    def body(rows_hbm, dst_hbm, zeros_hbm, parts_hbm, *scr):
        idxs = scr[0:K]
        buf = scr[K:2 * K]
        acc = scr[2 * K]
        sld = scr[2 * K + 1:3 * K + 1]
        ssc = scr[3 * K + 1:4 * K + 1]
        cid = lax.axis_index("c")
        sid = lax.axis_index("s")
        r0 = sid * rows_per_tile
        pltpu.sync_copy(zeros_hbm.at[pl.ds(r0, rows_per_tile)],
                        acc.at[pl.ds(r0, rows_per_tile)])
        plsc.subcore_barrier()

        base0 = (cid * NSUB + sid) * per_w

        def fire_load(slot, i):
            sl = pl.ds(base0 + i * chunk, chunk)
            pltpu.async_copy(dst_hbm.at[sl], idxs[slot], sld[slot])
            pltpu.async_copy(rows_hbm.at[sl], buf[slot], sld[slot])

        def wait_load(slot):
            pltpu.make_async_copy(dst_hbm.at[pl.ds(0, chunk)],
                                  idxs[slot], sld[slot]).wait()
            pltpu.make_async_copy(rows_hbm.at[pl.ds(0, chunk)], buf[slot],
                                  sld[slot]).wait()

        def wait_sc(slot):
            pltpu.make_async_copy(buf[slot], acc.at[idxs[slot]],
                                  ssc[slot]).wait()

        for p in range(K - 1):
            fire_load(p, p)

        def group(g, carry):
            for b in range(K):
                i = g * K + b
                wait_load(b)
                pltpu.async_copy(buf[b], acc.at[idxs[b]], ssc[b], add=True)
                j = i + K - 1
                s2 = (b + K - 1) % K

                @pl.when(j < iters)
                def _prefetch():
                    @pl.when(i > 0)
                    def _drain():
                        wait_sc(s2)
                    fire_load(s2, j)
            return carry

        lax.fori_loop(0, iters // K, group, 0)
        for b in range(K):
            wait_sc(b)
        plsc.subcore_barrier()
        pltpu.sync_copy(acc.at[pl.ds(r0, rows_per_tile)],
                        parts_hbm.at[cid, pl.ds(r0, rows_per_tile)])

    mesh = plsc.VectorSubcoreMesh(core_axis_name="c", subcore_axis_name="s")
    scratch = [pltpu.VMEM((chunk,), jnp.int32)] * K
    scratch += [pltpu.VMEM((chunk, H), jnp.float32)] * K
    scratch += [pltpu.VMEM_SHARED((n_nodes_pad, H), jnp.float32)]
    scratch += [pltpu.SemaphoreType.DMA] * (2 * K)
    return pl.kernel(
        body,
        out_type=jax.ShapeDtypeStruct((NCORES, n_nodes_pad, H), jnp.float32),
        mesh=mesh,
        scratch_types=scratch,
    )


def _edge_body(g_ref, a_ref, w1c_ref, w2_ref, b2_ref, gam_ref, bet_ref,
               enew_ref, eout_ref):
    a = a_ref[...]
    t = g_ref[...] + jnp.dot(a, w1c_ref[...], preferred_element_type=jnp.float32)
    h = jnp.maximum(t, 0.0)
    y = jnp.dot(h, w2_ref[...], preferred_element_type=jnp.float32) + b2_ref[...]
    m = jnp.mean(y, axis=-1, keepdims=True)
    d = y - m
    v = jnp.mean(d * d, axis=-1, keepdims=True)
    e = d * lax.rsqrt(v + 1e-5) * gam_ref[...] + bet_ref[...]
    enew_ref[...] = e
    eout_ref[...] = a + e


def _edge_mlp(g, attr, w1c, w2, b2, gam, bet, be=2000):
    n = g.shape[0]
    out = [jax.ShapeDtypeStruct((n, H), jnp.float32)] * 2
    row = pl.BlockSpec((be, H), lambda i: (i, 0))
    full = pl.BlockSpec((H, H), lambda i: (0, 0))
    vec = pl.BlockSpec((1, H), lambda i: (0, 0))
    return pl.pallas_call(
        _edge_body,
        grid=(n // be,),
        in_specs=[row, row, full, full, vec, vec, vec],
        out_specs=[row, row],
        out_shape=out,
    )(g, attr, w1c, w2, b2, gam, bet)


def _node_body(x_ref, pm_ref, pw_ref, w1_ref, b1_ref, w2_ref, b2_ref,
               gam_ref, bet_ref, out_ref):
    x = x_ref[...]
    aggm = pm_ref[0] + pm_ref[1]
    aggw = pw_ref[0] + pw_ref[1]
    cat = jnp.concatenate([x, aggm, aggw], axis=-1)
    t = jnp.dot(cat, w1_ref[...], preferred_element_type=jnp.float32) + b1_ref[...]
    h = jnp.maximum(t, 0.0)
    y = jnp.dot(h, w2_ref[...], preferred_element_type=jnp.float32) + b2_ref[...]
    m = jnp.mean(y, axis=-1, keepdims=True)
    d = y - m
    v = jnp.mean(d * d, axis=-1, keepdims=True)
    out_ref[...] = x + d * lax.rsqrt(v + 1e-5) * gam_ref[...] + bet_ref[...]


def _node_mlp(x, parts_m, parts_w, w1, b1, w2, b2, gam, bet, bn=2000):
    n = x.shape[0]
    row = pl.BlockSpec((bn, H), lambda i: (i, 0))
    prow = pl.BlockSpec((NCORES, bn, H), lambda i: (0, i, 0))
    vec = pl.BlockSpec((1, H), lambda i: (0, 0))
    return pl.pallas_call(
        _node_body,
        grid=(n // bn,),
        in_specs=[row, prow, prow,
                  pl.BlockSpec((3 * H, H), lambda i: (0, 0)), vec,
                  pl.BlockSpec((H, H), lambda i: (0, 0)), vec, vec, vec],
        out_specs=row,
        out_shape=jax.ShapeDtypeStruct((n, H), jnp.float32),
    )(x, parts_m, parts_w, w1, b1, w2, b2, gam, bet)


def kernel(x, edge_index, edge_attr, edge_world_index, edge_world_attr,
           emb_W1, emb_b1, emb_W2, emb_b2, emb_g, emb_be,
           ewb_W1, ewb_b1, ewb_W2, ewb_b2, ewb_g, ewb_be,
           nb_W1, nb_b1, nb_W2, nb_b2, nb_g, nb_be):
    n, _ = x.shape
    e = edge_attr.shape[0]
    ew = edge_world_attr.shape[0]

    wcat = jnp.concatenate(
        [emb_W1[0:H], emb_W1[H:2 * H], ewb_W1[0:H], ewb_W1[H:2 * H]], axis=1)
    zcol = jnp.zeros((H,), jnp.float32)
    bcat = jnp.concatenate([zcol, emb_b1, zcol, ewb_b1]).reshape(1, 4 * H)
    pam, pbm, paw, pbw = _precompute_tables(x, wcat, bcat)

    cm, cw, kslots = 40, 40, 5
    src_m, dst_m = edge_index[0], edge_index[1]
    src_w, dst_w = edge_world_index[0], edge_world_index[1]

    g_m = _make_gather_sum(e, cm, kslots)(pam, pbm, src_m, dst_m)
    g_w = _make_gather_sum(ew, cw, kslots)(paw, pbw, src_w, dst_w)

    v1 = lambda a: a.reshape(1, H)
    enew_m, eout_m = _edge_mlp(g_m, edge_attr, emb_W1[2 * H:], emb_W2,
                               v1(emb_b2), v1(emb_g), v1(emb_be))
    enew_w, eout_w = _edge_mlp(g_w, edge_world_attr, ewb_W1[2 * H:], ewb_W2,
                               v1(ewb_b2), v1(ewb_g), v1(ewb_be))

    n_pad = ((n + NSUB * 8 - 1) // (NSUB * 8)) * (NSUB * 8)
    zeros_nh = jnp.zeros((n_pad, H), jnp.float32)
    parts_m = _make_segment_sum(e, n_pad, cm, kslots)(enew_m, dst_m, zeros_nh)
    parts_w = _make_segment_sum(ew, n_pad, cw, kslots)(enew_w, dst_w, zeros_nh)

    x_out = _node_mlp(x, parts_m, parts_w, nb_W1, v1(nb_b1), nb_W2,
                      v1(nb_b2), v1(nb_g), v1(nb_be))
    return (x_out, eout_m, eout_w)
